# baseline (speedup 1.0000x reference)
import functools

import jax
import jax.numpy as jnp
from jax import lax
from jax.experimental import pallas as pl
from jax.experimental.pallas import tpu as pltpu
from jax.experimental.pallas import tpu_sc as plsc

N = 10000
E = 320000
F = 128
NW = 32
NPT = 313
NPAD = NW * NPT
EPW = E // NW
CG = 80
NCH_G = EPW // CG
CS = 128
FLS = 1024
STG = FLS + 16
CAP = EPW + FLS
BE = 2000
NEG = float("-inf")
IMIN = -2147483647

_mesh = plsc.VectorSubcoreMesh(core_axis_name="c", subcore_axis_name="s")


def _wid():
    return lax.axis_index("s") * 2 + lax.axis_index("c")


@functools.partial(
    pl.kernel,
    out_type=(
        jax.ShapeDtypeStruct((NW * NW * CAP,), jnp.int32),
        jax.ShapeDtypeStruct((NW * NW * 16,), jnp.int32),
    ),
    mesh=_mesh,
    scratch_types=[
        pltpu.VMEM((EPW,), jnp.int32),
        pltpu.VMEM((NW * STG,), jnp.int32),
        pltpu.VMEM((NW * 16,), jnp.int32),
        pltpu.VMEM((NW * 16,), jnp.int32),
        pltpu.VMEM((16,), jnp.int32),
    ],
)
def _bucketize(dst_hbm, lists_hbm, counts_hbm, dstv, stg, cnts, tots, ctmp):
    p = _wid()
    lane = lax.iota(jnp.int32, 16)
    inv = jnp.float32(1.0 / NPT)
    pltpu.sync_copy(dst_hbm.at[pl.ds(pl.multiple_of(p * EPW, 8), EPW)], dstv)

    def zinit(i, _):
        stg[pl.ds(i * 16, 16)] = jnp.zeros((16,), jnp.int32)
        return 0

    lax.fori_loop(0, NW * STG // 16, zinit, 0)
    zero16 = jnp.zeros((16,), jnp.int32)
    for o in range(NW):
        cnts[pl.ds(o * 16, 16)] = zero16
        tots[pl.ds(o * 16, 16)] = zero16

    def group(i, _):
        dv = dstv[pl.ds(i * 16, 16)]
        ov = (dv.astype(jnp.float32) * inv).astype(jnp.int32)
        r = dv - ov * NPT
        ov = ov + jnp.where(r >= NPT, 1, 0) - jnp.where(r < 0, 1, 0)
        loc = dv - ov * NPT
        packed = lax.shift_left(loc, 19) | (p * EPW + i * 16 + lane)
        for j in range(16):
            o = ov[j]
            pk = packed[j]
            cv = cnts[pl.ds(pl.multiple_of(o * 16, 8), 16)]
            c = cv[0]
            stg[pl.ds(o * STG + c, 16)] = zero16 + pk
            cnts[pl.ds(pl.multiple_of(o * 16, 8), 16)] = cv + 1

            @pl.when(c + 1 >= FLS)
            def _():
                tv = tots[pl.ds(pl.multiple_of(o * 16, 8), 16)]
                t = pl.multiple_of(tv[0], 8)
                pltpu.sync_copy(
                    stg.at[pl.ds(pl.multiple_of(o * STG, 8), FLS)],
                    lists_hbm.at[pl.ds(pl.multiple_of((p * NW + o) * CAP + t, 8), FLS)])
                cnts[pl.ds(pl.multiple_of(o * 16, 8), 16)] = zero16
                tots[pl.ds(pl.multiple_of(o * 16, 8), 16)] = tv + FLS
        return 0

    lax.fori_loop(0, EPW // 16, group, 0)
    for o in range(NW):
        cv = cnts[pl.ds(o * 16, 16)]
        tv = tots[pl.ds(o * 16, 16)]
        t = pl.multiple_of(tv[0], 8)
        pltpu.sync_copy(stg.at[pl.ds(o * STG, FLS)],
                        lists_hbm.at[pl.ds(pl.multiple_of((p * NW + o) * CAP + t, 8), FLS)])
        ctmp[...] = cv + tv
        pltpu.sync_copy(ctmp, counts_hbm.at[pl.ds(pl.multiple_of((p * NW + o) * 16, 8), 16)])


CG2 = 80
NFULL = EPW // CG2

@functools.partial(
    pl.kernel,
    out_type=(
        jax.ShapeDtypeStruct((E, F), jnp.int32),
        jax.ShapeDtypeStruct((E, F), jnp.int32),
    ),
    mesh=_mesh,
    scratch_types=[
        pltpu.VMEM((EPW,), jnp.int32),
        pltpu.VMEM((EPW,), jnp.int32),
        pltpu.VMEM((CG2, F), jnp.int32),
        pltpu.VMEM((CG2, F), jnp.int32),
        pltpu.VMEM((CG2, F), jnp.int32),
        pltpu.VMEM((CG2, F), jnp.int32),
    ] + [pltpu.SemaphoreType.DMA] * 8,
)
def _gather(src_hbm, dst_hbm, c_hbm, gs_hbm, gd_hbm,
            srcv, dstv, bs0, bs1, bd0, bd1,
            ss0, ss1, sd0, sd1, ts0, ts1, td0, td1):
    w = _wid()
    base = w * EPW
    pltpu.sync_copy(src_hbm.at[pl.ds(pl.multiple_of(base, 8), EPW)], srcv)
    pltpu.sync_copy(dst_hbm.at[pl.ds(pl.multiple_of(base, 8), EPW)], dstv)

    def issue(k, bs, bd, sems, semd):
        pltpu.make_async_copy(c_hbm.at[srcv.at[pl.ds(k * CG2, CG2)]], bs, sems).start()
        pltpu.make_async_copy(c_hbm.at[dstv.at[pl.ds(k * CG2, CG2)]], bd, semd).start()

    def waitpair(bs, bd, sems, semd):
        pltpu.make_async_copy(c_hbm.at[pl.ds(0, CG2)], bs, sems).wait()
        pltpu.make_async_copy(c_hbm.at[pl.ds(0, CG2)], bd, semd).wait()

    def store_start(k, bs, bd, sts, std):
        off = pl.ds(pl.multiple_of(base + k * CG2, 8), CG2)
        pltpu.make_async_copy(bs, gs_hbm.at[off, :], sts).start()
        pltpu.make_async_copy(bd, gd_hbm.at[off, :], std).start()

    def store_wait(bs, bd, sts, std):
        pltpu.make_async_copy(bs, gs_hbm.at[pl.ds(0, CG2), :], sts).wait()
        pltpu.make_async_copy(bd, gd_hbm.at[pl.ds(0, CG2), :], std).wait()

    issue(0, bs0, bd0, ss0, sd0)
    issue(1, bs1, bd1, ss1, sd1)

    def pair(i2, _):
        a = i2 * 2
        waitpair(bs0, bd0, ss0, sd0)
        store_start(a, bs0, bd0, ts0, td0)

        waitpair(bs1, bd1, ss1, sd1)
        store_start(a + 1, bs1, bd1, ts1, td1)

        store_wait(bs0, bd0, ts0, td0)

        @pl.when(a + 2 < NFULL)
        def _():
            issue(a + 2, bs0, bd0, ss0, sd0)

        store_wait(bs1, bd1, ts1, td1)

        @pl.when(a + 3 < NFULL)
        def _():
            issue(a + 3, bs1, bd1, ss1, sd1)

        return 0

    lax.fori_loop(0, NFULL // 2, pair, 0)
    waitpair(bs0, bd0, ss0, sd0)
    store_start(NFULL - 1, bs0, bd0, ts0, td0)
    store_wait(bs0, bd0, ts0, td0)


@functools.partial(
    pl.kernel,
    out_type=jax.ShapeDtypeStruct((NPAD * F,), jnp.float32),
    mesh=_mesh,
    scratch_types=[
        pltpu.VMEM((CS,), jnp.int32),
        pltpu.VMEM((CS,), jnp.int32),
        pltpu.VMEM((CS, F), jnp.float32),
        pltpu.VMEM((NPT * F,), jnp.float32),
        pltpu.VMEM((16,), jnp.int32),
        pltpu.SemaphoreType.DMA,
    ],
)
def _scatter_max(m_hbm, lists_hbm, counts_hbm, agg_hbm,
                 idbuf, pkbuf, rows, acc, cbuf, sem):
    o = _wid()

    def ainit(i, _):
        acc[pl.ds(i * 16, 16)] = jnp.full((16,), NEG, jnp.float32)
        return 0

    lax.fori_loop(0, NPT * F // 16, ainit, 0)

    def producer(p, _):
        pltpu.sync_copy(
            counts_hbm.at[pl.ds(pl.multiple_of((p * NW + o) * 16, 8), 16)], cbuf)
        cw = cbuf[...][0]
        bbase = (p * NW + o) * CAP

        def chunk(k, _):
            cb = k * CS
            pltpu.sync_copy(
                lists_hbm.at[pl.ds(pl.multiple_of(bbase + cb, 8), CS)], pkbuf)
            for g in range(CS // 16):
                idbuf[pl.ds(g * 16, 16)] = pkbuf[pl.ds(g * 16, 16)] & 0x7FFFF
            pltpu.async_copy(m_hbm.at[idbuf], rows, sem).wait()
            for g in range(CS // 16):
                locv = lax.shift_right_logical(pkbuf[pl.ds(g * 16, 16)], 19)
                for j in range(16):
                    jj = g * 16 + j
                    off = locv[j]

                    @pl.when(cb + jj < cw)
                    def _():
                        for c in range(F // 16):
                            ap = pl.ds(off * F + c * 16, 16)
                            acc[ap] = jnp.maximum(acc[ap], rows[jj, pl.ds(c * 16, 16)])
            return 0

        lax.fori_loop(0, (cw + CS - 1) // CS, chunk, 0)
        return 0

    lax.fori_loop(0, NW, producer, 0)

    def fin(i, _):
        v = acc[pl.ds(i * 16, 16)]
        acc[pl.ds(i * 16, 16)] = jnp.where(v == NEG, 0.0, v)
        return 0

    lax.fori_loop(0, NPT * F // 16, fin, 0)
    pltpu.sync_copy(acc, agg_hbm.at[pl.ds(pl.multiple_of(o * NPT * F, 8), NPT * F)])


def _pack_bf(f):
    u = lax.bitcast_convert_type(f, jnp.uint32)
    r = u + jnp.uint32(0x7FFF) + (lax.shift_right_logical(u, jnp.uint32(16)) & jnp.uint32(1))
    lo = lax.shift_right_logical(r[:, :F // 2], jnp.uint32(16))
    hi = r[:, F // 2:] & jnp.uint32(0xFFFF0000)
    return lax.bitcast_convert_type(lo | hi, jnp.int32)


def _unpack_bf(w):
    u = lax.bitcast_convert_type(w, jnp.uint32)
    lo = lax.bitcast_convert_type(lax.shift_left(u, jnp.uint32(16)), jnp.float32)
    hi = lax.bitcast_convert_type(u & jnp.uint32(0xFFFF0000), jnp.float32)
    return jnp.concatenate([lo, hi], axis=1)


def _dense_kernel(x_ref, w_ref, b_ref, o_ref):
    o_ref[...] = jnp.dot(x_ref[...], w_ref[...],
                         preferred_element_type=jnp.float32) + b_ref[...]


def _td_kernel(f_ref, pp_ref, w1a_ref, fb_ref):
    f = f_ref[...]
    pp = pp_ref[...]
    a = jnp.dot(jnp.maximum(f * pp[0:1, :] + pp[1:2, :], 0.0), w1a_ref[...],
                preferred_element_type=jnp.float32)
    fb_ref[...] = jnp.concatenate([_pack_bf(f), _pack_bf(a)], axis=1)


def _mid_td_kernel(agg_ref, prev_ref, pp_ref, w1a_ref, f_ref, fb_ref, *, has_prev):
    pp = pp_ref[...]
    h = agg_ref[...]
    if has_prev:
        h = h + prev_ref[...]
    f = jnp.maximum(h * pp[0:1, :] + pp[1:2, :], 0.0)
    f_ref[...] = f
    a = jnp.dot(jnp.maximum(f * pp[2:3, :] + pp[3:4, :], 0.0), w1a_ref[...],
                preferred_element_type=jnp.float32)
    fb_ref[...] = jnp.concatenate([_pack_bf(f), _pack_bf(a)], axis=1)


def _edge_mlp_kernel(gs_ref, gd_ref, pp_ref, w1b_ref, w2_ref, o_ref):
    pp = pp_ref[...]
    gd = gd_ref[...]
    d = _unpack_bf(gs_ref[...][:, :F // 2]) - _unpack_bf(gd[:, :F // 2])
    u = jnp.maximum(d * pp[0:1, :] + pp[1:2, :], 0.0)
    h1 = (jnp.dot(u, w1b_ref[...], preferred_element_type=jnp.float32)
          + _unpack_bf(gd[:, F // 2:]))
    h2 = jnp.maximum(h1 * pp[2:3, :] + pp[3:4, :], 0.0)
    o_ref[...] = jnp.dot(h2, w2_ref[...], preferred_element_type=jnp.float32)


def _row_spec(rows, cols):
    return pl.BlockSpec((rows, cols), lambda i: (i, 0))


def _rep_spec(rows, cols):
    return pl.BlockSpec((rows, cols), lambda i: (0, 0))


def _call_dense(x, w, b):
    n = x.shape[0]
    return pl.pallas_call(
        _dense_kernel,
        grid=(n // 1000,),
        in_specs=[_row_spec(1000, x.shape[1]), _rep_spec(*w.shape), _rep_spec(1, w.shape[1])],
        out_specs=_row_spec(1000, w.shape[1]),
        out_shape=jax.ShapeDtypeStruct((n, w.shape[1]), jnp.float32),
    )(x, w, b)


def _call_td(f, pp, w1a):
    return pl.pallas_call(
        _td_kernel,
        grid=(N // 1000,),
        in_specs=[_row_spec(1000, F), _rep_spec(2, F), _rep_spec(F, F)],
        out_specs=_row_spec(1000, F),
        out_shape=jax.ShapeDtypeStruct((N, F), jnp.int32),
    )(f, pp, w1a)


def _call_mid_td(agg, prev, pp, w1a, has_prev):
    fn = functools.partial(_mid_td_kernel, has_prev=has_prev)
    return pl.pallas_call(
        fn,
        grid=(N // 1000,),
        in_specs=[_row_spec(1000, F), _row_spec(1000, F), _rep_spec(4, F),
                  _rep_spec(F, F)],
        out_specs=[_row_spec(1000, F), _row_spec(1000, F)],
        out_shape=[jax.ShapeDtypeStruct((N, F), jnp.float32),
                   jax.ShapeDtypeStruct((N, F), jnp.int32)],
    )(agg, prev, pp, w1a)


def _call_edge_mlp(gs, gd, pp, w1b, w2):
    return pl.pallas_call(
        _edge_mlp_kernel,
        grid=(E // BE,),
        in_specs=[_row_spec(BE, F), _row_spec(BE, F),
                  _rep_spec(4, F), _rep_spec(F, F), _rep_spec(F, F)],
        out_specs=_row_spec(BE, F),
        out_shape=jax.ShapeDtypeStruct((E, F), jnp.float32),
    )(gs, gd, pp, w1b, w2)


def kernel(x, edge_index, w_av, b_av, lpp_bn1_g, lpp_bn1_b, lpp_fc1_w,
           lpp_bn2_g, lpp_bn2_b, lpp_fc2_w, mid_bn_g, mid_bn_b, fc_w, fc_b):
    c = 1.0 / jnp.sqrt(jnp.float32(1.0 + 1e-5))
    src = edge_index[0]
    dst = edge_index[1]
    lists, counts = _bucketize(dst)

    g = _call_dense(x.reshape(N, 2 * F), w_av.T, b_av.reshape(1, F))

    feats = g
    ctab = _call_td(g, jnp.stack([c * lpp_bn1_g[0, :F], lpp_bn1_b[0, :F]]),
                    lpp_fc1_w[0][:, :F].T)

    for i in range(4):
        gs, gd = _gather(src, dst, ctab)
        ppe = jnp.stack([c * lpp_bn1_g[i, F:], lpp_bn1_b[i, F:],
                         c * lpp_bn2_g[i], lpp_bn2_b[i]])
        m = _call_edge_mlp(gs, gd, ppe, lpp_fc1_w[i][:, F:].T, lpp_fc2_w[i].T)
        agg = _scatter_max(m, lists, counts).reshape(NPAD, F)[:N]
        if i < 3:
            ppm = jnp.stack([c * mid_bn_g[i], mid_bn_b[i],
                             c * lpp_bn1_g[i + 1, :F], lpp_bn1_b[i + 1, :F]])
            feats, ctab = _call_mid_td(agg, feats, ppm, lpp_fc1_w[i + 1][:, :F].T,
                                       has_prev=(i > 0))

    g4 = agg + feats
    wpad = jnp.zeros((F, F), jnp.float32).at[:, :2].set(fc_w.T)
    bpad = jnp.zeros((1, F), jnp.float32).at[0, :2].set(fc_b)
    out = _call_dense(g4, wpad, bpad)
    return out[:, :2]


# speedup vs baseline: 1.0371x; 1.0371x over previous
import functools

import jax
import jax.numpy as jnp
from jax import lax
from jax.experimental import pallas as pl
from jax.experimental.pallas import tpu as pltpu
from jax.experimental.pallas import tpu_sc as plsc

N = 10000
E = 320000
F = 128
NW = 32
NPT = 313
NPAD = NW * NPT
EPW = E // NW
CG = 80
NCH_G = EPW // CG
CS = 128
FLS = 1024
STG = FLS + 16
CAP = EPW + FLS
BE = 2000
NEG = float("-inf")
IMIN = -2147483647

_mesh = plsc.VectorSubcoreMesh(core_axis_name="c", subcore_axis_name="s")


def _wid():
    return lax.axis_index("s") * 2 + lax.axis_index("c")


@functools.partial(
    pl.kernel,
    out_type=(
        jax.ShapeDtypeStruct((NW * NW * CAP,), jnp.int32),
        jax.ShapeDtypeStruct((NW * NW * 16,), jnp.int32),
    ),
    mesh=_mesh,
    scratch_types=[
        pltpu.VMEM((EPW,), jnp.int32),
        pltpu.VMEM((NW * STG,), jnp.int32),
        pltpu.VMEM((NW * 16,), jnp.int32),
        pltpu.VMEM((NW * 16,), jnp.int32),
        pltpu.VMEM((16,), jnp.int32),
    ],
)
def _bucketize(dst_hbm, lists_hbm, counts_hbm, dstv, stg, cnts, tots, ctmp):
    p = _wid()
    lane = lax.iota(jnp.int32, 16)
    inv = jnp.float32(1.0 / NPT)
    pltpu.sync_copy(dst_hbm.at[pl.ds(pl.multiple_of(p * EPW, 8), EPW)], dstv)

    def zinit(i, _):
        stg[pl.ds(i * 16, 16)] = jnp.zeros((16,), jnp.int32)
        return 0

    lax.fori_loop(0, NW * STG // 16, zinit, 0)
    zero16 = jnp.zeros((16,), jnp.int32)
    for o in range(NW):
        cnts[pl.ds(o * 16, 16)] = zero16
        tots[pl.ds(o * 16, 16)] = zero16

    def group(i, _):
        dv = dstv[pl.ds(i * 16, 16)]
        ov = (dv.astype(jnp.float32) * inv).astype(jnp.int32)
        r = dv - ov * NPT
        ov = ov + jnp.where(r >= NPT, 1, 0) - jnp.where(r < 0, 1, 0)
        loc = dv - ov * NPT
        packed = lax.shift_left(loc, 19) | (p * EPW + i * 16 + lane)
        for j in range(16):
            o = ov[j]
            pk = packed[j]
            cv = cnts[pl.ds(pl.multiple_of(o * 16, 8), 16)]
            c = cv[0]
            stg[pl.ds(o * STG + c, 16)] = zero16 + pk
            cnts[pl.ds(pl.multiple_of(o * 16, 8), 16)] = cv + 1

            @pl.when(c + 1 >= FLS)
            def _():
                tv = tots[pl.ds(pl.multiple_of(o * 16, 8), 16)]
                t = pl.multiple_of(tv[0], 8)
                pltpu.sync_copy(
                    stg.at[pl.ds(pl.multiple_of(o * STG, 8), FLS)],
                    lists_hbm.at[pl.ds(pl.multiple_of((p * NW + o) * CAP + t, 8), FLS)])
                cnts[pl.ds(pl.multiple_of(o * 16, 8), 16)] = zero16
                tots[pl.ds(pl.multiple_of(o * 16, 8), 16)] = tv + FLS
        return 0

    lax.fori_loop(0, EPW // 16, group, 0)
    for o in range(NW):
        cv = cnts[pl.ds(o * 16, 16)]
        tv = tots[pl.ds(o * 16, 16)]
        t = pl.multiple_of(tv[0], 8)
        pltpu.sync_copy(stg.at[pl.ds(o * STG, FLS)],
                        lists_hbm.at[pl.ds(pl.multiple_of((p * NW + o) * CAP + t, 8), FLS)])
        ctmp[...] = cv + tv
        pltpu.sync_copy(ctmp, counts_hbm.at[pl.ds(pl.multiple_of((p * NW + o) * 16, 8), 16)])


CG2 = 40
EHALF = EPW // 2
NHALF = EHALF // CG2

@functools.partial(
    pl.kernel,
    out_type=(
        jax.ShapeDtypeStruct((E, F), jnp.int32),
        jax.ShapeDtypeStruct((E, F), jnp.int32),
    ),
    mesh=_mesh,
    scratch_types=[
        pltpu.VMEM_SHARED((N, F), jnp.int32),
        pltpu.VMEM((EHALF,), jnp.int32),
        pltpu.VMEM((EHALF,), jnp.int32),
        pltpu.VMEM((CG2, F), jnp.int32),
        pltpu.VMEM((CG2, F), jnp.int32),
        pltpu.VMEM((CG2, F), jnp.int32),
        pltpu.VMEM((CG2, F), jnp.int32),
    ] + [pltpu.SemaphoreType.DMA] * 8,
)
def _gather(src_hbm, dst_hbm, c_hbm, gs_hbm, gd_hbm,
            shc, srcv, dstv, bs0, bs1, bd0, bd1,
            ss0, ss1, sd0, sd1, ts0, ts1, td0, td1):
    w = _wid()
    sid = lax.axis_index("s")

    @pl.when(sid == 0)
    def _():
        pltpu.sync_copy(c_hbm, shc)

    plsc.subcore_barrier()

    def issue(k, bs, bd, sems, semd):
        pltpu.make_async_copy(shc.at[srcv.at[pl.ds(k * CG2, CG2)]], bs, sems).start()
        pltpu.make_async_copy(shc.at[dstv.at[pl.ds(k * CG2, CG2)]], bd, semd).start()

    def waitpair(bs, bd, sems, semd):
        pltpu.make_async_copy(shc.at[pl.ds(0, CG2)], bs, sems).wait()
        pltpu.make_async_copy(shc.at[pl.ds(0, CG2)], bd, semd).wait()

    for half in range(2):
        base = w * EPW + half * EHALF
        pltpu.sync_copy(src_hbm.at[pl.ds(pl.multiple_of(base, 8), EHALF)], srcv)
        pltpu.sync_copy(dst_hbm.at[pl.ds(pl.multiple_of(base, 8), EHALF)], dstv)

        def store_start(k, bs, bd, sts, std):
            off = pl.ds(pl.multiple_of(base + k * CG2, 8), CG2)
            pltpu.make_async_copy(bs, gs_hbm.at[off, :], sts).start()
            pltpu.make_async_copy(bd, gd_hbm.at[off, :], std).start()

        def store_wait(bs, bd, sts, std):
            pltpu.make_async_copy(bs, gs_hbm.at[pl.ds(0, CG2), :], sts).wait()
            pltpu.make_async_copy(bd, gd_hbm.at[pl.ds(0, CG2), :], std).wait()

        issue(0, bs0, bd0, ss0, sd0)
        issue(1, bs1, bd1, ss1, sd1)

        def pair(i2, _):
            a = i2 * 2
            waitpair(bs0, bd0, ss0, sd0)
            store_start(a, bs0, bd0, ts0, td0)

            waitpair(bs1, bd1, ss1, sd1)
            store_start(a + 1, bs1, bd1, ts1, td1)

            store_wait(bs0, bd0, ts0, td0)

            @pl.when(a + 2 < NHALF)
            def _():
                issue(a + 2, bs0, bd0, ss0, sd0)

            store_wait(bs1, bd1, ts1, td1)

            @pl.when(a + 3 < NHALF)
            def _():
                issue(a + 3, bs1, bd1, ss1, sd1)

            return 0

        lax.fori_loop(0, NHALF // 2, pair, 0)
        waitpair(bs0, bd0, ss0, sd0)
        store_start(NHALF - 1, bs0, bd0, ts0, td0)
        store_wait(bs0, bd0, ts0, td0)


@functools.partial(
    pl.kernel,
    out_type=jax.ShapeDtypeStruct((NPAD * F,), jnp.float32),
    mesh=_mesh,
    scratch_types=[
        pltpu.VMEM((CS,), jnp.int32),
        pltpu.VMEM((CS,), jnp.int32),
        pltpu.VMEM((CS, F), jnp.float32),
        pltpu.VMEM((NPT * F,), jnp.float32),
        pltpu.VMEM((16,), jnp.int32),
        pltpu.SemaphoreType.DMA,
    ],
)
def _scatter_max(m_hbm, lists_hbm, counts_hbm, agg_hbm,
                 idbuf, pkbuf, rows, acc, cbuf, sem):
    o = _wid()

    def ainit(i, _):
        acc[pl.ds(i * 16, 16)] = jnp.full((16,), NEG, jnp.float32)
        return 0

    lax.fori_loop(0, NPT * F // 16, ainit, 0)

    def producer(p, _):
        pltpu.sync_copy(
            counts_hbm.at[pl.ds(pl.multiple_of((p * NW + o) * 16, 8), 16)], cbuf)
        cw = cbuf[...][0]
        bbase = (p * NW + o) * CAP

        def chunk(k, _):
            cb = k * CS
            pltpu.sync_copy(
                lists_hbm.at[pl.ds(pl.multiple_of(bbase + cb, 8), CS)], pkbuf)
            for g in range(CS // 16):
                idbuf[pl.ds(g * 16, 16)] = pkbuf[pl.ds(g * 16, 16)] & 0x7FFFF
            pltpu.async_copy(m_hbm.at[idbuf], rows, sem).wait()
            for g in range(CS // 16):
                locv = lax.shift_right_logical(pkbuf[pl.ds(g * 16, 16)], 19)
                for j in range(16):
                    jj = g * 16 + j
                    off = locv[j]

                    @pl.when(cb + jj < cw)
                    def _():
                        for c in range(F // 16):
                            ap = pl.ds(off * F + c * 16, 16)
                            acc[ap] = jnp.maximum(acc[ap], rows[jj, pl.ds(c * 16, 16)])
            return 0

        lax.fori_loop(0, (cw + CS - 1) // CS, chunk, 0)
        return 0

    lax.fori_loop(0, NW, producer, 0)

    def fin(i, _):
        v = acc[pl.ds(i * 16, 16)]
        acc[pl.ds(i * 16, 16)] = jnp.where(v == NEG, 0.0, v)
        return 0

    lax.fori_loop(0, NPT * F // 16, fin, 0)
    pltpu.sync_copy(acc, agg_hbm.at[pl.ds(pl.multiple_of(o * NPT * F, 8), NPT * F)])


def _pack_bf(f):
    u = lax.bitcast_convert_type(f, jnp.uint32)
    r = u + jnp.uint32(0x7FFF) + (lax.shift_right_logical(u, jnp.uint32(16)) & jnp.uint32(1))
    lo = lax.shift_right_logical(r[:, :F // 2], jnp.uint32(16))
    hi = r[:, F // 2:] & jnp.uint32(0xFFFF0000)
    return lax.bitcast_convert_type(lo | hi, jnp.int32)


def _unpack_bf(w):
    u = lax.bitcast_convert_type(w, jnp.uint32)
    lo = lax.bitcast_convert_type(lax.shift_left(u, jnp.uint32(16)), jnp.float32)
    hi = lax.bitcast_convert_type(u & jnp.uint32(0xFFFF0000), jnp.float32)
    return jnp.concatenate([lo, hi], axis=1)


def _dense_kernel(x_ref, w_ref, b_ref, o_ref):
    o_ref[...] = jnp.dot(x_ref[...], w_ref[...],
                         preferred_element_type=jnp.float32) + b_ref[...]


def _td_kernel(f_ref, pp_ref, w1a_ref, fb_ref):
    f = f_ref[...]
    pp = pp_ref[...]
    a = jnp.dot(jnp.maximum(f * pp[0:1, :] + pp[1:2, :], 0.0), w1a_ref[...],
                preferred_element_type=jnp.float32)
    fb_ref[...] = jnp.concatenate([_pack_bf(f), _pack_bf(a)], axis=1)


def _mid_td_kernel(agg_ref, prev_ref, pp_ref, w1a_ref, f_ref, fb_ref, *, has_prev):
    pp = pp_ref[...]
    h = agg_ref[...]
    if has_prev:
        h = h + prev_ref[...]
    f = jnp.maximum(h * pp[0:1, :] + pp[1:2, :], 0.0)
    f_ref[...] = f
    a = jnp.dot(jnp.maximum(f * pp[2:3, :] + pp[3:4, :], 0.0), w1a_ref[...],
                preferred_element_type=jnp.float32)
    fb_ref[...] = jnp.concatenate([_pack_bf(f), _pack_bf(a)], axis=1)


def _edge_mlp_kernel(gs_ref, gd_ref, pp_ref, w1b_ref, w2_ref, o_ref):
    pp = pp_ref[...]
    gd = gd_ref[...]
    d = _unpack_bf(gs_ref[...][:, :F // 2]) - _unpack_bf(gd[:, :F // 2])
    u = jnp.maximum(d * pp[0:1, :] + pp[1:2, :], 0.0)
    h1 = (jnp.dot(u, w1b_ref[...], preferred_element_type=jnp.float32)
          + _unpack_bf(gd[:, F // 2:]))
    h2 = jnp.maximum(h1 * pp[2:3, :] + pp[3:4, :], 0.0)
    o_ref[...] = jnp.dot(h2, w2_ref[...], preferred_element_type=jnp.float32)


def _row_spec(rows, cols):
    return pl.BlockSpec((rows, cols), lambda i: (i, 0))


def _rep_spec(rows, cols):
    return pl.BlockSpec((rows, cols), lambda i: (0, 0))


def _call_dense(x, w, b):
    n = x.shape[0]
    return pl.pallas_call(
        _dense_kernel,
        grid=(n // 1000,),
        in_specs=[_row_spec(1000, x.shape[1]), _rep_spec(*w.shape), _rep_spec(1, w.shape[1])],
        out_specs=_row_spec(1000, w.shape[1]),
        out_shape=jax.ShapeDtypeStruct((n, w.shape[1]), jnp.float32),
    )(x, w, b)


def _call_td(f, pp, w1a):
    return pl.pallas_call(
        _td_kernel,
        grid=(N // 1000,),
        in_specs=[_row_spec(1000, F), _rep_spec(2, F), _rep_spec(F, F)],
        out_specs=_row_spec(1000, F),
        out_shape=jax.ShapeDtypeStruct((N, F), jnp.int32),
    )(f, pp, w1a)


def _call_mid_td(agg, prev, pp, w1a, has_prev):
    fn = functools.partial(_mid_td_kernel, has_prev=has_prev)
    return pl.pallas_call(
        fn,
        grid=(N // 1000,),
        in_specs=[_row_spec(1000, F), _row_spec(1000, F), _rep_spec(4, F),
                  _rep_spec(F, F)],
        out_specs=[_row_spec(1000, F), _row_spec(1000, F)],
        out_shape=[jax.ShapeDtypeStruct((N, F), jnp.float32),
                   jax.ShapeDtypeStruct((N, F), jnp.int32)],
    )(agg, prev, pp, w1a)


def _call_edge_mlp(gs, gd, pp, w1b, w2):
    return pl.pallas_call(
        _edge_mlp_kernel,
        grid=(E // BE,),
        in_specs=[_row_spec(BE, F), _row_spec(BE, F),
                  _rep_spec(4, F), _rep_spec(F, F), _rep_spec(F, F)],
        out_specs=_row_spec(BE, F),
        out_shape=jax.ShapeDtypeStruct((E, F), jnp.float32),
    )(gs, gd, pp, w1b, w2)


def kernel(x, edge_index, w_av, b_av, lpp_bn1_g, lpp_bn1_b, lpp_fc1_w,
           lpp_bn2_g, lpp_bn2_b, lpp_fc2_w, mid_bn_g, mid_bn_b, fc_w, fc_b):
    c = 1.0 / jnp.sqrt(jnp.float32(1.0 + 1e-5))
    src = edge_index[0]
    dst = edge_index[1]
    lists, counts = _bucketize(dst)

    g = _call_dense(x.reshape(N, 2 * F), w_av.T, b_av.reshape(1, F))

    feats = g
    ctab = _call_td(g, jnp.stack([c * lpp_bn1_g[0, :F], lpp_bn1_b[0, :F]]),
                    lpp_fc1_w[0][:, :F].T)

    for i in range(4):
        gs, gd = _gather(src, dst, ctab)
        ppe = jnp.stack([c * lpp_bn1_g[i, F:], lpp_bn1_b[i, F:],
                         c * lpp_bn2_g[i], lpp_bn2_b[i]])
        m = _call_edge_mlp(gs, gd, ppe, lpp_fc1_w[i][:, F:].T, lpp_fc2_w[i].T)
        agg = _scatter_max(m, lists, counts).reshape(NPAD, F)[:N]
        if i < 3:
            ppm = jnp.stack([c * mid_bn_g[i], mid_bn_b[i],
                             c * lpp_bn1_g[i + 1, :F], lpp_bn1_b[i + 1, :F]])
            feats, ctab = _call_mid_td(agg, feats, ppm, lpp_fc1_w[i + 1][:, :F].T,
                                       has_prev=(i > 0))

    g4 = agg + feats
    wpad = jnp.zeros((F, F), jnp.float32).at[:, :2].set(fc_w.T)
    bpad = jnp.zeros((1, F), jnp.float32).at[0, :2].set(fc_b)
    out = _call_dense(g4, wpad, bpad)
    return out[:, :2]
